# baseline (speedup 1.0000x reference)
import functools

import jax
import jax.numpy as jnp
from jax import lax
from jax.experimental import pallas as pl
from jax.experimental.pallas import tpu as pltpu
from jax.experimental.pallas import tpu_sc as plsc

_LANES = 16


def _selector_sc(B, S, D, dtype):
    mesh = plsc.VectorSubcoreMesh(
        core_axis_name="c", subcore_axis_name="s", num_cores=1, num_subcores=1
    )

    @functools.partial(
        pl.kernel,
        mesh=mesh,
        out_type=jax.ShapeDtypeStruct((B, D), dtype),
        scratch_types=[
            pltpu.VMEM((_LANES,), jnp.int32),
            pltpu.VMEM((_LANES, D), dtype),
            pltpu.SemaphoreType.DMA,
        ],
    )
    def gather_kernel(x_hbm, idx_hbm, out_hbm, idx_v, rows_v, sem):
        pltpu.sync_copy(idx_hbm, idx_v.at[pl.ds(0, B)])
        lane = lax.iota(jnp.int32, _LANES)
        idx_v[...] = jnp.where(lane < B, idx_v[...] + lane * S, 0)
        pltpu.async_copy(x_hbm.at[idx_v], rows_v, sem).wait()
        pltpu.sync_copy(rows_v.at[pl.ds(0, B)], out_hbm)

    return gather_kernel


def kernel(x, idx):
    B, S, D = x.shape
    assert B <= _LANES
    x_flat = x.reshape(B * S, D)
    return _selector_sc(B, S, D, x.dtype)(x_flat, idx.astype(jnp.int32))


# speedup vs baseline: 1.1256x; 1.1256x over previous
import functools

import jax
import jax.numpy as jnp
from jax import lax
from jax.experimental import pallas as pl
from jax.experimental.pallas import tpu as pltpu
from jax.experimental.pallas import tpu_sc as plsc


def _selector_scs(B, S, D, dtype):
    mesh = plsc.ScalarSubcoreMesh(axis_name="c", num_cores=1)

    @functools.partial(
        pl.kernel,
        mesh=mesh,
        out_type=jax.ShapeDtypeStruct((B, D), dtype),
        scratch_types=[
            pltpu.SMEM((B,), jnp.int32),
            pltpu.SemaphoreType.DMA,
        ],
    )
    def gather_kernel(x_hbm, gidx_hbm, out_hbm, idx_s, sem):
        pltpu.sync_copy(gidx_hbm, idx_s)
        copies = []
        for b in range(B):
            copies.append(
                pltpu.make_async_copy(
                    x_hbm.at[pl.ds(idx_s[b], 1)], out_hbm.at[pl.ds(b, 1)], sem
                )
            )
        for c in copies:
            c.start()
        for c in copies:
            c.wait()

    return gather_kernel


def kernel(x, idx):
    B, S, D = x.shape
    x_flat = x.reshape(B * S, D)
    gidx = idx.astype(jnp.int32) + jnp.arange(B, dtype=jnp.int32) * S
    return _selector_scs(B, S, D, x.dtype)(x_flat, gidx)


# speedup vs baseline: 1.1322x; 1.0059x over previous
import functools

import jax
import jax.numpy as jnp
from jax.experimental import pallas as pl
from jax.experimental.pallas import tpu as pltpu
from jax.experimental.pallas import tpu_sc as plsc


def _selector_scs(B, S, D, dtype):
    mesh = plsc.ScalarSubcoreMesh(axis_name="c", num_cores=1)

    @functools.partial(
        pl.kernel,
        mesh=mesh,
        out_type=jax.ShapeDtypeStruct((B, D), dtype),
        scratch_types=[
            pltpu.SMEM((B,), jnp.int32),
            pltpu.SemaphoreType.DMA,
        ],
    )
    def gather_kernel(x_hbm, idx_hbm, out_hbm, idx_s, sem):
        pltpu.sync_copy(idx_hbm, idx_s)
        copies = []
        for b in range(B):
            copies.append(
                pltpu.make_async_copy(
                    x_hbm.at[pl.ds(idx_s[b] + b * S, 1)],
                    out_hbm.at[pl.ds(b, 1)],
                    sem,
                )
            )
        for c in copies:
            c.start()
        for c in copies:
            c.wait()

    return gather_kernel


def kernel(x, idx):
    B, S, D = x.shape
    x_flat = x.reshape(B * S, D)
    return _selector_scs(B, S, D, x.dtype)(x_flat, idx.astype(jnp.int32))
